# baseline (speedup 1.0000x reference)
import jax
import jax.numpy as jnp
import numpy as np
from jax import lax
from jax.experimental import pallas as pl
from jax.experimental.pallas import tpu as pltpu
from jax.experimental.pallas import tpu_sc as plsc

N_WORDS = 1000
B, S = 16, 2048
GROUP_DIMS = (512, 256, 256)
N_CB = 4
OUT_D = sum(GROUP_DIMS)
N_TAB = 12

NC, NS, L = 2, 16, 16
NW = NC * NS
TOK = B * S
T_PER_W = TOK // NW

CHUNK = 32
N_CHUNK = T_PER_W // CHUNK


def _accum_bf(acc, tmp, nh):
  def body(i, carry):
    for h in range(nh):
      for jj in range(4):
        sl = pl.ds(jj * 32, 32)
        acc[i, h, sl] = acc[i, h, sl] + tmp[i, h, sl]
    return carry

  lax.fori_loop(0, CHUNK, body, 0)


def _accum_cvt(acc, tmp, stage, nh):
  even = lax.iota(jnp.int32, L) * 2
  odd = even + 1

  def body(i, carry):
    for h in range(nh):
      for jj in range(4):
        s = acc[i, h, pl.ds(jj * 32, 32)] + tmp[i, h, pl.ds(jj * 32, 32)]
        a, b = plsc.unpack(s, format=plsc.PackFormat.INTERLEAVED)
        row = stage.at[i, h, pl.ds(jj * 32, 32)]
        plsc.store_scatter(row, [even], a)
        plsc.store_scatter(row, [odd], b)
    return carry

  lax.fori_loop(0, CHUNK, body, 0)


def _sc_body(x5, sos, t00, t01, t02, t03, t10, t11, t12, t13, t20, t21,
             t22, t23, out, idx_v, a0a, a0b, m0a, m0b, g0a, g0b, a1a, a1b,
             m1a, m1b, g1a, g1b, sos_v, sA0a, sA0b, sT0a, sT0b, sO0a, sO0b,
             sA1a, sA1b, sT1a, sT1b, sO1a, sO1b):
  group_tabs = ((t00, t01, t02, t03), (t10, t11, t12, t13),
                (t20, t21, t22, t23))
  wid = lax.axis_index("s") * NC + lax.axis_index("c")
  b = wid // 2
  b_hi = b // 8
  b_lo = b % 8
  half = wid % 2
  s0 = half * T_PER_W

  pltpu.sync_copy(x5.at[:, b_hi, pl.ds(half * 8, 8), b_lo, :], idx_v)

  pltpu.sync_copy(sos, sos_v)

  @pl.when(half == 0)
  def _():
    pltpu.sync_copy(sos_v, out.at[0, b_hi, :, b_lo, :])

  def run_group(tabs, nh, h0, jbase, accs, tmps, stages, sA, sT, sO):
    def gidx(c, j):
      return idx_v.at[jbase + j, c // 4, pl.ds((c % 4) * CHUNK, CHUNK)]

    def gather(j, c, buf, sem):
      pltpu.async_copy(tabs[j].at[gidx(c, j)], buf, sem)

    def wait_gather(buf, sem):
      pltpu.make_async_copy(tabs[0].at[pl.ds(0, CHUNK)], buf, sem).wait()

    def out_dst(c):
      return out.at[pl.ds(1 + s0 + c * CHUNK, CHUNK), b_hi,
                    pl.ds(h0, nh), b_lo, :]

    def wait_out(p):
      pltpu.make_async_copy(stages[p], out_dst(0), sO[p]).wait()

    def do_chunk(c, p):
      q = 1 - p
      acc = accs[p]

      @pl.when(c < N_CHUNK - 1)
      def _():
        gather(0, c + 1, accs[q], sA[q])

      wait_gather(acc, sA[p])
      wait_gather(tmps[p], sT[p])
      gather(2, c, tmps[q], sT[q])
      _accum_bf(acc, tmps[p], nh)
      wait_gather(tmps[q], sT[q])
      gather(3, c, tmps[p], sT[p])
      _accum_bf(acc, tmps[q], nh)
      wait_gather(tmps[p], sT[p])

      @pl.when(c < N_CHUNK - 1)
      def _():
        gather(1, c + 1, tmps[q], sT[q])

      @pl.when(c >= 2)
      def _():
        wait_out(p)

      _accum_cvt(acc, tmps[p], stages[p], nh)
      pltpu.async_copy(stages[p], out_dst(c), sO[p])

    gather(0, 0, accs[0], sA[0])
    gather(1, 0, tmps[0], sT[0])

    def pair_body(c2, carry):
      do_chunk(2 * c2, 0)
      do_chunk(2 * c2 + 1, 1)
      return carry

    lax.fori_loop(0, N_CHUNK // 2, pair_body, 0)
    return wait_out

  w0 = run_group(group_tabs[0], 4, 0, 0, (a0a, a0b), (m0a, m0b), (g0a, g0b),
                 (sA0a, sA0b), (sT0a, sT0b), (sO0a, sO0b))
  w1 = run_group(group_tabs[1], 2, 4, 4, (a1a, a1b), (m1a, m1b), (g1a, g1b),
                 (sA1a, sA1b), (sT1a, sT1b), (sO1a, sO1b))
  w1(0)
  w1(1)
  w2 = run_group(group_tabs[2], 2, 6, 8, (a1a, a1b), (m1a, m1b), (g1a, g1b),
                 (sA1a, sA1b), (sT1a, sT1b), (sO1a, sO1b))
  w2(0)
  w2(1)
  w0(0)
  w0(1)


def kernel(x, sos, table_0_0, table_0_1, table_0_2, table_0_3, table_1_0,
           table_1_1, table_1_2, table_1_3, table_2_0, table_2_1, table_2_2,
           table_2_3):
  x5 = (x.transpose((2, 0, 1)).reshape(N_TAB, 2, 8, 16, 128)
        .transpose((0, 1, 3, 2, 4)))
  sos_2d = sos.reshape(8, 128)
  tabs = [
      t.astype(jnp.bfloat16).reshape(N_WORDS + 1, d // 128, 128)
      for t, d in zip(
          (table_0_0, table_0_1, table_0_2, table_0_3, table_1_0, table_1_1,
           table_1_2, table_1_3, table_2_0, table_2_1, table_2_2, table_2_3),
          (512,) * 4 + (256,) * 8)
  ]

  mesh = plsc.VectorSubcoreMesh(
      core_axis_name="c", subcore_axis_name="s", num_cores=NC,
      num_subcores=NS)
  kfn = pl.kernel(
      _sc_body,
      out_type=jax.ShapeDtypeStruct((S + 1, 2, 8, 8, 128), jnp.float32),
      mesh=mesh,
      compiler_params=pltpu.CompilerParams(
          use_tc_tiling_on_sc=False, needs_layout_passes=False),
      scratch_types=[
          pltpu.VMEM((N_TAB, 8, 128), jnp.int32),
          pltpu.VMEM((CHUNK, 4, 128), jnp.bfloat16),
          pltpu.VMEM((CHUNK, 4, 128), jnp.bfloat16),
          pltpu.VMEM((CHUNK, 4, 128), jnp.bfloat16),
          pltpu.VMEM((CHUNK, 4, 128), jnp.bfloat16),
          pltpu.VMEM((CHUNK, 4, 128), jnp.float32),
          pltpu.VMEM((CHUNK, 4, 128), jnp.float32),
          pltpu.VMEM((CHUNK, 2, 128), jnp.bfloat16),
          pltpu.VMEM((CHUNK, 2, 128), jnp.bfloat16),
          pltpu.VMEM((CHUNK, 2, 128), jnp.bfloat16),
          pltpu.VMEM((CHUNK, 2, 128), jnp.bfloat16),
          pltpu.VMEM((CHUNK, 2, 128), jnp.float32),
          pltpu.VMEM((CHUNK, 2, 128), jnp.float32),
          pltpu.VMEM((8, 128), jnp.float32),
      ] + [pltpu.SemaphoreType.DMA] * 12,
  )
  out5 = kfn(x5, sos_2d, *tabs)
  return out5.transpose((1, 3, 0, 2, 4)).reshape(B, S + 1, OUT_D)
